# Initial kernel scaffold; baseline (speedup 1.0000x reference)
import jax
import jax.numpy as jnp
from jax.experimental import pallas as pl


def kernel(z, edge_index):
    raise NotImplementedError("write your pallas kernel here")


# speedup vs baseline: 3.4772x; 3.4772x over previous
import functools

import jax
import jax.numpy as jnp
from jax import lax
from jax.experimental import pallas as pl
from jax.experimental.pallas import tpu as pltpu
from jax.experimental.pallas import tpu_sc as plsc

N_NODES = 10000
N_EDGES = 320000
D = 128

NC = 2
NS = 16
NW = NC * NS
E_PER_TILE = N_EDGES // NW
CHUNK = 80
N_CHUNKS = E_PER_TILE // CHUNK
GROUPS = CHUNK // 16

_mesh = plsc.VectorSubcoreMesh(core_axis_name="c", subcore_axis_name="s")

_GATHER_DIMS = lax.GatherDimensionNumbers(
    offset_dims=(), collapsed_slice_dims=(0,), start_index_map=(0,))


def _shuffle(v, perm):
    return lax.gather(v, perm[:, None], _GATHER_DIMS, slice_sizes=(1,),
                      mode=lax.GatherScatterMode.PROMISE_IN_BOUNDS)


@functools.partial(
    pl.kernel,
    mesh=_mesh,
    out_type=jax.ShapeDtypeStruct((N_EDGES,), jnp.float32),
    scratch_types=[
        pltpu.VMEM((E_PER_TILE,), jnp.int32),
        pltpu.VMEM((E_PER_TILE,), jnp.int32),
        pltpu.VMEM((CHUNK, D), jnp.float32),
        pltpu.VMEM((CHUNK, D), jnp.float32),
        pltpu.VMEM((E_PER_TILE,), jnp.float32),
        pltpu.SemaphoreType.DMA,
    ],
)
def _decode(z_hbm, row_hbm, col_hbm, out_hbm, idx_r, idx_c, rows_r, rows_c,
            out_v, sem):
    wid = lax.axis_index("s") * NC + lax.axis_index("c")
    base = wid * E_PER_TILE
    pltpu.sync_copy(row_hbm.at[pl.ds(base, E_PER_TILE)], idx_r)
    pltpu.sync_copy(col_hbm.at[pl.ds(base, E_PER_TILE)], idx_c)

    lane = lax.iota(jnp.int32, 16)

    def chunk_body(ci, carry):
        off = ci * CHUNK
        cr = pltpu.async_copy(z_hbm.at[idx_r.at[pl.ds(off, CHUNK)]], rows_r, sem)
        cc = pltpu.async_copy(z_hbm.at[idx_c.at[pl.ds(off, CHUNK)]], rows_c, sem)
        cr.wait()
        cc.wait()

        def group_body(g, c2):
            e0 = g * 16
            v = jnp.zeros((16,), jnp.float32)
            for e in range(16):
                ec = e0 + e
                acc = rows_r[ec, pl.ds(0, 16)] * rows_c[ec, pl.ds(0, 16)]
                for k in range(1, D // 16):
                    acc = acc + (rows_r[ec, pl.ds(16 * k, 16)]
                                 * rows_c[ec, pl.ds(16 * k, 16)])
                for sh in (8, 4, 2, 1):
                    perm = jnp.arange(16, dtype=jnp.int32) ^ sh
                    acc = acc + _shuffle(acc, perm)
                v = jnp.where(lane == e, acc, v)
            out_v[pl.ds(off + e0, 16)] = 1.0 / (1.0 + jnp.exp(-v))
            return c2

        lax.fori_loop(0, GROUPS, group_body, 0)
        return carry

    lax.fori_loop(0, N_CHUNKS, chunk_body, 0)
    pltpu.sync_copy(out_v, out_hbm.at[pl.ds(base, E_PER_TILE)])


def kernel(z, edge_index):
    row = edge_index[0].astype(jnp.int32)
    col = edge_index[1].astype(jnp.int32)
    return _decode(z, row, col)


# speedup vs baseline: 5.0483x; 1.4518x over previous
import functools

import jax
import jax.numpy as jnp
from jax import lax
from jax.experimental import pallas as pl
from jax.experimental.pallas import tpu as pltpu
from jax.experimental.pallas import tpu_sc as plsc

N_NODES = 10000
N_EDGES = 320000
D = 128

NC = 2
NS = 16
NW = NC * NS
E_PER_TILE = N_EDGES // NW
CHUNK = 80
N_CHUNKS = E_PER_TILE // CHUNK
GROUPS = CHUNK // 16

_mesh = plsc.VectorSubcoreMesh(core_axis_name="c", subcore_axis_name="s")

_GATHER_DIMS = lax.GatherDimensionNumbers(
    offset_dims=(), collapsed_slice_dims=(0,), start_index_map=(0,))


def _shuffle(v, perm):
    return lax.gather(v, perm[:, None], _GATHER_DIMS, slice_sizes=(1,),
                      mode=lax.GatherScatterMode.PROMISE_IN_BOUNDS)


@functools.partial(
    pl.kernel,
    mesh=_mesh,
    out_type=jax.ShapeDtypeStruct((N_EDGES,), jnp.float32),
    scratch_types=[
        pltpu.VMEM((E_PER_TILE,), jnp.int32),
        pltpu.VMEM((E_PER_TILE,), jnp.int32),
        pltpu.VMEM((CHUNK, D), jnp.float32),
        pltpu.VMEM((CHUNK, D), jnp.float32),
        pltpu.VMEM((E_PER_TILE,), jnp.float32),
        pltpu.SemaphoreType.DMA,
    ],
)
def _decode(z_hbm, row_hbm, col_hbm, out_hbm, idx_r, idx_c, rows_r, rows_c,
            out_v, sem):
    wid = lax.axis_index("s") * NC + lax.axis_index("c")
    base = wid * E_PER_TILE
    pltpu.sync_copy(row_hbm.at[pl.ds(base, E_PER_TILE)], idx_r)
    pltpu.sync_copy(col_hbm.at[pl.ds(base, E_PER_TILE)], idx_c)

    lane = lax.iota(jnp.int32, 16)

    def chunk_body(ci, carry):
        off = ci * CHUNK
        cr = pltpu.async_copy(z_hbm.at[idx_r.at[pl.ds(off, CHUNK)]], rows_r, sem)
        cc = pltpu.async_copy(z_hbm.at[idx_c.at[pl.ds(off, CHUNK)]], rows_c, sem)
        cr.wait()
        cc.wait()

        def group_body(g, c2):
            e0 = g * 16

            def quad_body(q, v):
                for u in range(4):
                    e = 4 * q + u
                    ec = e0 + e
                    p = [rows_r[ec, pl.ds(16 * k, 16)]
                         * rows_c[ec, pl.ds(16 * k, 16)]
                         for k in range(D // 16)]
                    s = ((p[0] + p[1]) + (p[2] + p[3])) \
                        + ((p[4] + p[5]) + (p[6] + p[7]))
                    for sh in (8, 4, 2, 1):
                        perm = jnp.arange(16, dtype=jnp.int32) ^ sh
                        s = s + _shuffle(s, perm)
                    v = jnp.where(lane == e, s, v)
                return v

            v = lax.fori_loop(0, 4, quad_body, jnp.zeros((16,), jnp.float32))
            out_v[pl.ds(off + e0, 16)] = 1.0 / (1.0 + jnp.exp(-v))
            return c2

        lax.fori_loop(0, GROUPS, group_body, 0)
        return carry

    lax.fori_loop(0, N_CHUNKS, chunk_body, 0)
    pltpu.sync_copy(out_v, out_hbm.at[pl.ds(base, E_PER_TILE)])


def kernel(z, edge_index):
    row = edge_index[0].astype(jnp.int32)
    col = edge_index[1].astype(jnp.int32)
    return _decode(z, row, col)


# speedup vs baseline: 8.7562x; 1.7345x over previous
import functools

import jax
import jax.numpy as jnp
from jax import lax
from jax.experimental import pallas as pl
from jax.experimental.pallas import tpu as pltpu
from jax.experimental.pallas import tpu_sc as plsc

N_NODES = 10000
N_EDGES = 320000
D = 128

NC = 2
NS = 16
NW = NC * NS
E_PER_TILE = N_EDGES // NW
CHUNK = 80
N_CHUNKS = E_PER_TILE // CHUNK
GROUPS = CHUNK // 16

_mesh = plsc.VectorSubcoreMesh(core_axis_name="c", subcore_axis_name="s")

_GATHER_DIMS = lax.GatherDimensionNumbers(
    offset_dims=(), collapsed_slice_dims=(0,), start_index_map=(0,))


def _shuffle(v, perm):
    return lax.gather(v, perm[:, None], _GATHER_DIMS, slice_sizes=(1,),
                      mode=lax.GatherScatterMode.PROMISE_IN_BOUNDS)


@functools.partial(
    pl.kernel,
    mesh=_mesh,
    out_type=jax.ShapeDtypeStruct((N_EDGES,), jnp.float32),
    scratch_types=[
        pltpu.VMEM((E_PER_TILE,), jnp.int32),
        pltpu.VMEM((E_PER_TILE,), jnp.int32),
        pltpu.VMEM((CHUNK, D), jnp.float32),
        pltpu.VMEM((CHUNK, D), jnp.float32),
        pltpu.VMEM((CHUNK, D), jnp.float32),
        pltpu.VMEM((CHUNK, D), jnp.float32),
        pltpu.VMEM((E_PER_TILE,), jnp.float32),
        pltpu.SemaphoreType.DMA,
        pltpu.SemaphoreType.DMA,
    ],
)
def _decode(z_hbm, row_hbm, col_hbm, out_hbm, idx_r, idx_c,
            rr0, rc0, rr1, rc1, out_v, sem0, sem1):
    wid = lax.axis_index("s") * NC + lax.axis_index("c")
    base = wid * E_PER_TILE
    pltpu.sync_copy(row_hbm.at[pl.ds(base, E_PER_TILE)], idx_r)
    pltpu.sync_copy(col_hbm.at[pl.ds(base, E_PER_TILE)], idx_c)

    lane = lax.iota(jnp.int32, 16)

    def start(ci, rr, rc, sem):
        o = ci * CHUNK
        pltpu.async_copy(z_hbm.at[idx_r.at[pl.ds(o, CHUNK)]], rr, sem)
        pltpu.async_copy(z_hbm.at[idx_c.at[pl.ds(o, CHUNK)]], rc, sem)

    def drain(rr, rc, sem):
        pltpu.make_async_copy(z_hbm.at[pl.ds(0, CHUNK)], rr, sem).wait()
        pltpu.make_async_copy(z_hbm.at[pl.ds(0, CHUNK)], rc, sem).wait()

    def compute(off, rr, rc):
        def group_body(g, c2):
            e0 = g * 16

            def quad_body(q, v):
                for u in range(4):
                    e = 4 * q + u
                    ec = e0 + e
                    p = [rr[ec, pl.ds(16 * k, 16)] * rc[ec, pl.ds(16 * k, 16)]
                         for k in range(D // 16)]
                    s = ((p[0] + p[1]) + (p[2] + p[3])) \
                        + ((p[4] + p[5]) + (p[6] + p[7]))
                    for sh in (8, 4, 2, 1):
                        perm = jnp.arange(16, dtype=jnp.int32) ^ sh
                        s = s + _shuffle(s, perm)
                    v = jnp.where(lane == e, s, v)
                return v

            v = lax.fori_loop(0, 4, quad_body, jnp.zeros((16,), jnp.float32))
            out_v[pl.ds(off + e0, 16)] = 1.0 / (1.0 + jnp.exp(-v))
            return c2

        lax.fori_loop(0, GROUPS, group_body, 0)

    start(0, rr0, rc0, sem0)

    def pipe_body(i, carry):
        ci = 2 * i
        start(ci + 1, rr1, rc1, sem1)
        drain(rr0, rc0, sem0)
        compute(ci * CHUNK, rr0, rc0)
        start(ci + 2, rr0, rc0, sem0)
        drain(rr1, rc1, sem1)
        compute((ci + 1) * CHUNK, rr1, rc1)
        return carry

    lax.fori_loop(0, (N_CHUNKS - 1) // 2, pipe_body, 0)
    drain(rr0, rc0, sem0)
    compute((N_CHUNKS - 1) * CHUNK, rr0, rc0)
    pltpu.sync_copy(out_v, out_hbm.at[pl.ds(base, E_PER_TILE)])


def kernel(z, edge_index):
    row = edge_index[0].astype(jnp.int32)
    col = edge_index[1].astype(jnp.int32)
    return _decode(z, row, col)


# speedup vs baseline: 9.1841x; 1.0489x over previous
import functools

import jax
import jax.numpy as jnp
from jax import lax
from jax.experimental import pallas as pl
from jax.experimental.pallas import tpu as pltpu
from jax.experimental.pallas import tpu_sc as plsc

N_NODES = 10000
N_EDGES = 320000
D = 128

NC = 2
NS = 16
NW = NC * NS
E_PER_TILE = N_EDGES // NW
CHUNK = 80
N_CHUNKS = E_PER_TILE // CHUNK
GROUPS = CHUNK // 16

_mesh = plsc.VectorSubcoreMesh(core_axis_name="c", subcore_axis_name="s")

_GATHER_DIMS = lax.GatherDimensionNumbers(
    offset_dims=(), collapsed_slice_dims=(0,), start_index_map=(0,))


def _shuffle(v, perm):
    return lax.gather(v, perm[:, None], _GATHER_DIMS, slice_sizes=(1,),
                      mode=lax.GatherScatterMode.PROMISE_IN_BOUNDS)


def _halves(u):
    lo = plsc.bitcast(u << jnp.uint32(16), jnp.float32)
    hi = plsc.bitcast(u & jnp.uint32(0xFFFF0000), jnp.float32)
    return lo, hi


@functools.partial(
    pl.kernel,
    mesh=_mesh,
    compiler_params=pltpu.CompilerParams(needs_layout_passes=False,
                                         use_tc_tiling_on_sc=False),
    out_type=jax.ShapeDtypeStruct((N_EDGES,), jnp.float32),
    scratch_types=[
        pltpu.VMEM((E_PER_TILE,), jnp.int32),
        pltpu.VMEM((E_PER_TILE,), jnp.int32),
        pltpu.VMEM((CHUNK, D // 2), jnp.uint32),
        pltpu.VMEM((CHUNK, D // 2), jnp.uint32),
        pltpu.VMEM((CHUNK, D // 2), jnp.uint32),
        pltpu.VMEM((CHUNK, D // 2), jnp.uint32),
        pltpu.VMEM((E_PER_TILE,), jnp.float32),
        pltpu.SemaphoreType.DMA,
        pltpu.SemaphoreType.DMA,
    ],
)
def _decode(z_hbm, row_hbm, col_hbm, out_hbm, idx_r, idx_c,
            rr0, rc0, rr1, rc1, out_v, sem0, sem1):
    wid = lax.axis_index("s") * NC + lax.axis_index("c")
    base = wid * E_PER_TILE
    pltpu.sync_copy(row_hbm.at[pl.ds(base, E_PER_TILE)], idx_r)
    pltpu.sync_copy(col_hbm.at[pl.ds(base, E_PER_TILE)], idx_c)

    lane = lax.iota(jnp.int32, 16)

    def start(ci, rr, rc, sem):
        o = ci * CHUNK
        pltpu.async_copy(z_hbm.at[idx_r.at[pl.ds(o, CHUNK)]], rr, sem)
        pltpu.async_copy(z_hbm.at[idx_c.at[pl.ds(o, CHUNK)]], rc, sem)

    def drain(rr, rc, sem):
        pltpu.make_async_copy(z_hbm.at[pl.ds(0, CHUNK)], rr, sem).wait()
        pltpu.make_async_copy(z_hbm.at[pl.ds(0, CHUNK)], rc, sem).wait()

    def compute(off, rr, rc):
        def group_body(g, c2):
            e0 = g * 16

            def quad_body(q, v):
                for u in range(4):
                    e = 4 * q + u
                    ec = e0 + e
                    p = []
                    for k in range(D // 32):
                        rlo, rhi = _halves(rr[ec, pl.ds(16 * k, 16)])
                        clo, chi = _halves(rc[ec, pl.ds(16 * k, 16)])
                        p.append(rlo * clo)
                        p.append(rhi * chi)
                    s = ((p[0] + p[1]) + (p[2] + p[3])) \
                        + ((p[4] + p[5]) + (p[6] + p[7]))
                    for sh in (8, 4, 2, 1):
                        perm = jnp.arange(16, dtype=jnp.int32) ^ sh
                        s = s + _shuffle(s, perm)
                    v = jnp.where(lane == e, s, v)
                return v

            v = lax.fori_loop(0, 4, quad_body, jnp.zeros((16,), jnp.float32))
            out_v[pl.ds(off + e0, 16)] = 1.0 / (1.0 + jnp.exp(-v))
            return c2

        lax.fori_loop(0, GROUPS, group_body, 0)

    start(0, rr0, rc0, sem0)

    def pipe_body(i, carry):
        ci = 2 * i
        start(ci + 1, rr1, rc1, sem1)
        drain(rr0, rc0, sem0)
        compute(ci * CHUNK, rr0, rc0)
        start(ci + 2, rr0, rc0, sem0)
        drain(rr1, rc1, sem1)
        compute((ci + 1) * CHUNK, rr1, rc1)
        return carry

    lax.fori_loop(0, (N_CHUNKS - 1) // 2, pipe_body, 0)
    drain(rr0, rc0, sem0)
    compute((N_CHUNKS - 1) * CHUNK, rr0, rc0)
    pltpu.sync_copy(out_v, out_hbm.at[pl.ds(base, E_PER_TILE)])


def kernel(z, edge_index):
    row = edge_index[0].astype(jnp.int32)
    col = edge_index[1].astype(jnp.int32)
    zp = lax.bitcast_convert_type(
        z.astype(jnp.bfloat16).reshape(N_NODES, D // 2, 2), jnp.uint32)
    return _decode(zp, row, col)


# speedup vs baseline: 11.7668x; 1.2812x over previous
import functools

import jax
import jax.numpy as jnp
from jax import lax
from jax.experimental import pallas as pl
from jax.experimental.pallas import tpu as pltpu
from jax.experimental.pallas import tpu_sc as plsc

N_NODES = 10000
N_EDGES = 320000
D = 128

NC = 2
NS = 16
NW = NC * NS
E_PER_TILE = N_EDGES // NW
CHUNK = 80
N_CHUNKS = E_PER_TILE // CHUNK
GROUPS = CHUNK // 16

_mesh = plsc.VectorSubcoreMesh(core_axis_name="c", subcore_axis_name="s")

_GATHER_DIMS = lax.GatherDimensionNumbers(
    offset_dims=(), collapsed_slice_dims=(0,), start_index_map=(0,))


def _shuffle(v, perm):
    return lax.gather(v, perm[:, None], _GATHER_DIMS, slice_sizes=(1,),
                      mode=lax.GatherScatterMode.PROMISE_IN_BOUNDS)


def _halves(u):
    lo = plsc.bitcast(u << jnp.uint32(16), jnp.float32)
    hi = plsc.bitcast(u, jnp.float32)
    return lo, hi


@functools.partial(
    pl.kernel,
    mesh=_mesh,
    compiler_params=pltpu.CompilerParams(needs_layout_passes=False,
                                         use_tc_tiling_on_sc=False),
    out_type=jax.ShapeDtypeStruct((N_EDGES,), jnp.float32),
    scratch_types=[
        pltpu.VMEM((E_PER_TILE,), jnp.int32),
        pltpu.VMEM((E_PER_TILE,), jnp.int32),
        pltpu.VMEM((CHUNK, D // 2), jnp.uint32),
        pltpu.VMEM((CHUNK, D // 2), jnp.uint32),
        pltpu.VMEM((CHUNK, D // 2), jnp.uint32),
        pltpu.VMEM((CHUNK, D // 2), jnp.uint32),
        pltpu.VMEM((E_PER_TILE,), jnp.float32),
        pltpu.SemaphoreType.DMA,
        pltpu.SemaphoreType.DMA,
    ],
)
def _decode(z_hbm, ei_hbm, out_hbm, idx_r, idx_c,
            rr0, rc0, rr1, rc1, out_v, sem0, sem1):
    wid = lax.axis_index("s") * NC + lax.axis_index("c")
    base = wid * E_PER_TILE
    pltpu.sync_copy(ei_hbm.at[0, pl.ds(base, E_PER_TILE)], idx_r)
    pltpu.sync_copy(ei_hbm.at[1, pl.ds(base, E_PER_TILE)], idx_c)

    lane = lax.iota(jnp.int32, 16)

    def start(ci, rr, rc, sem):
        o = ci * CHUNK
        pltpu.async_copy(z_hbm.at[idx_r.at[pl.ds(o, CHUNK)]], rr, sem)
        pltpu.async_copy(z_hbm.at[idx_c.at[pl.ds(o, CHUNK)]], rc, sem)

    def drain(rr, rc, sem):
        pltpu.make_async_copy(z_hbm.at[pl.ds(0, CHUNK)], rr, sem).wait()
        pltpu.make_async_copy(z_hbm.at[pl.ds(0, CHUNK)], rc, sem).wait()

    def compute(off, rr, rc):
        def group_body(g, c2):
            e0 = g * 16

            def quad_body(q, v):
                for u in range(4):
                    e = 4 * q + u
                    ec = e0 + e
                    p = []
                    for k in range(D // 32):
                        rlo, rhi = _halves(rr[ec, pl.ds(16 * k, 16)])
                        clo, chi = _halves(rc[ec, pl.ds(16 * k, 16)])
                        p.append(rlo * clo)
                        p.append(rhi * chi)
                    s = ((p[0] + p[1]) + (p[2] + p[3])) \
                        + ((p[4] + p[5]) + (p[6] + p[7]))
                    for sh in (8, 4, 2, 1):
                        perm = jnp.arange(16, dtype=jnp.int32) ^ sh
                        s = s + _shuffle(s, perm)
                    v = jnp.where(lane == e, s, v)
                return v

            v = lax.fori_loop(0, 4, quad_body, jnp.zeros((16,), jnp.float32))
            out_v[pl.ds(off + e0, 16)] = 1.0 / (1.0 + jnp.exp(-v))
            return c2

        lax.fori_loop(0, GROUPS, group_body, 0)

    start(0, rr0, rc0, sem0)

    def pipe_body(i, carry):
        ci = 2 * i
        start(ci + 1, rr1, rc1, sem1)
        drain(rr0, rc0, sem0)
        compute(ci * CHUNK, rr0, rc0)
        start(ci + 2, rr0, rc0, sem0)
        drain(rr1, rc1, sem1)
        compute((ci + 1) * CHUNK, rr1, rc1)
        return carry

    lax.fori_loop(0, (N_CHUNKS - 1) // 2, pipe_body, 0)
    drain(rr0, rc0, sem0)
    compute((N_CHUNKS - 1) * CHUNK, rr0, rc0)
    pltpu.sync_copy(out_v, out_hbm.at[pl.ds(base, E_PER_TILE)])


def _round_bf16_bits(u):
    return (u + jnp.uint32(0x7FFF) + ((u >> 16) & jnp.uint32(1))) >> 16


def kernel(z, edge_index):
    u = lax.bitcast_convert_type(z, jnp.uint32)
    ua = _round_bf16_bits(u[:, : D // 2])
    ub = _round_bf16_bits(u[:, D // 2:])
    zp = ua | (ub << 16)
    return _decode(zp, edge_index.astype(jnp.int32))


# speedup vs baseline: 12.0121x; 1.0208x over previous
import functools

import jax
import jax.numpy as jnp
from jax import lax
from jax.experimental import pallas as pl
from jax.experimental.pallas import tpu as pltpu
from jax.experimental.pallas import tpu_sc as plsc

N_NODES = 10000
N_EDGES = 320000
D = 128

NC = 2
NS = 16
NW = NC * NS
E_PER_TILE = N_EDGES // NW
CHUNK = 80
N_CHUNKS = E_PER_TILE // CHUNK
GROUPS = CHUNK // 16

_mesh = plsc.VectorSubcoreMesh(core_axis_name="c", subcore_axis_name="s")

_GATHER_DIMS = lax.GatherDimensionNumbers(
    offset_dims=(), collapsed_slice_dims=(0,), start_index_map=(0,))


def _shuffle(v, perm):
    return lax.gather(v, perm[:, None], _GATHER_DIMS, slice_sizes=(1,),
                      mode=lax.GatherScatterMode.PROMISE_IN_BOUNDS)


def _halves(u):
    lo = plsc.bitcast(u << jnp.uint32(16), jnp.float32)
    hi = plsc.bitcast(u, jnp.float32)
    return lo, hi


@functools.partial(
    pl.kernel,
    mesh=_mesh,
    compiler_params=pltpu.CompilerParams(needs_layout_passes=False,
                                         use_tc_tiling_on_sc=False),
    out_type=jax.ShapeDtypeStruct((N_EDGES,), jnp.float32),
    scratch_types=[
        pltpu.VMEM((E_PER_TILE,), jnp.int32),
        pltpu.VMEM((E_PER_TILE,), jnp.int32),
        pltpu.VMEM((CHUNK, D // 2), jnp.uint32),
        pltpu.VMEM((CHUNK, D // 2), jnp.uint32),
        pltpu.VMEM((CHUNK, D // 2), jnp.uint32),
        pltpu.VMEM((CHUNK, D // 2), jnp.uint32),
        pltpu.VMEM((E_PER_TILE,), jnp.float32),
        pltpu.SemaphoreType.DMA,
        pltpu.SemaphoreType.DMA,
    ],
)
def _decode(z_hbm, ei_hbm, out_hbm, idx_r, idx_c,
            rr0, rc0, rr1, rc1, out_v, sem0, sem1):
    wid = lax.axis_index("s") * NC + lax.axis_index("c")
    base = wid * E_PER_TILE
    pltpu.sync_copy(ei_hbm.at[0, pl.ds(base, E_PER_TILE)], idx_r)
    pltpu.sync_copy(ei_hbm.at[1, pl.ds(base, E_PER_TILE)], idx_c)

    lane = lax.iota(jnp.int32, 16)

    def start(ci, rr, rc, sem):
        o = ci * CHUNK
        pltpu.async_copy(z_hbm.at[idx_r.at[pl.ds(o, CHUNK)]], rr, sem)
        pltpu.async_copy(z_hbm.at[idx_c.at[pl.ds(o, CHUNK)]], rc, sem)

    def drain(rr, rc, sem):
        pltpu.make_async_copy(z_hbm.at[pl.ds(0, CHUNK)], rr, sem).wait()
        pltpu.make_async_copy(z_hbm.at[pl.ds(0, CHUNK)], rc, sem).wait()

    def compute(off, rr, rc):
        def group_body(g, c2):
            e0 = g * 16

            def quad_body(q, v):
                ss = []
                for u in range(4):
                    ec = e0 + 4 * q + u
                    p = []
                    for k in range(D // 32):
                        rlo, rhi = _halves(rr[ec, pl.ds(16 * k, 16)])
                        clo, chi = _halves(rc[ec, pl.ds(16 * k, 16)])
                        p.append(rlo * clo)
                        p.append(rhi * chi)
                    ss.append(((p[0] + p[1]) + (p[2] + p[3]))
                              + ((p[4] + p[5]) + (p[6] + p[7])))
                f = [s + _shuffle(s, lane ^ 8) for s in ss]
                m01 = jnp.where(lane < 8, f[0], f[1])
                m23 = jnp.where(lane < 8, f[2], f[3])
                m01 = m01 + _shuffle(m01, lane ^ 4)
                m23 = m23 + _shuffle(m23, lane ^ 4)
                mall = jnp.where((lane & 4) == 0, m01, m23)
                mall = mall + _shuffle(mall, lane ^ 2)
                mall = mall + _shuffle(mall, lane ^ 1)
                lu = lane & 3
                t = _shuffle(mall, ((lu & 1) << 3) | ((lu & 2) << 1))
                return jnp.where((lane >> 2) == q, t, v)

            v = lax.fori_loop(0, 4, quad_body, jnp.zeros((16,), jnp.float32))
            out_v[pl.ds(off + e0, 16)] = 1.0 / (1.0 + jnp.exp(-v))
            return c2

        lax.fori_loop(0, GROUPS, group_body, 0)

    start(0, rr0, rc0, sem0)

    def pipe_body(i, carry):
        ci = 2 * i
        start(ci + 1, rr1, rc1, sem1)
        drain(rr0, rc0, sem0)
        compute(ci * CHUNK, rr0, rc0)
        start(ci + 2, rr0, rc0, sem0)
        drain(rr1, rc1, sem1)
        compute((ci + 1) * CHUNK, rr1, rc1)
        return carry

    lax.fori_loop(0, (N_CHUNKS - 1) // 2, pipe_body, 0)
    drain(rr0, rc0, sem0)
    compute((N_CHUNKS - 1) * CHUNK, rr0, rc0)
    pltpu.sync_copy(out_v, out_hbm.at[pl.ds(base, E_PER_TILE)])


def _round_bf16_bits(u):
    return (u + jnp.uint32(0x7FFF) + ((u >> 16) & jnp.uint32(1))) >> 16


def kernel(z, edge_index):
    u = lax.bitcast_convert_type(z, jnp.uint32)
    ua = _round_bf16_bits(u[:, : D // 2])
    ub = _round_bf16_bits(u[:, D // 2:])
    zp = ua | (ub << 16)
    return _decode(zp, edge_index.astype(jnp.int32))


# speedup vs baseline: 14.0724x; 1.1715x over previous
import functools

import jax
import jax.numpy as jnp
from jax import lax
from jax.experimental import pallas as pl
from jax.experimental.pallas import tpu as pltpu
from jax.experimental.pallas import tpu_sc as plsc

N_NODES = 10000
N_EDGES = 320000
D = 128

NC = 2
NS = 16
NW = NC * NS
E_PER_TILE = N_EDGES // NW
CHUNK = 80
N_CHUNKS = E_PER_TILE // CHUNK
GROUPS = CHUNK // 16

_mesh = plsc.VectorSubcoreMesh(core_axis_name="c", subcore_axis_name="s")

_GATHER_DIMS = lax.GatherDimensionNumbers(
    offset_dims=(), collapsed_slice_dims=(0,), start_index_map=(0,))


def _shuffle(v, perm):
    return lax.gather(v, perm[:, None], _GATHER_DIMS, slice_sizes=(1,),
                      mode=lax.GatherScatterMode.PROMISE_IN_BOUNDS)


def _halves(u):
    lo = plsc.bitcast(u << jnp.uint32(16), jnp.float32)
    hi = plsc.bitcast(u, jnp.float32)
    return lo, hi


@functools.partial(
    pl.kernel,
    mesh=_mesh,
    compiler_params=pltpu.CompilerParams(needs_layout_passes=False,
                                         use_tc_tiling_on_sc=False),
    out_type=jax.ShapeDtypeStruct((N_EDGES,), jnp.float32),
    scratch_types=[
        pltpu.VMEM((E_PER_TILE,), jnp.int32),
        pltpu.VMEM((E_PER_TILE,), jnp.int32),
        pltpu.VMEM((CHUNK, D // 2), jnp.uint32),
        pltpu.VMEM((CHUNK, D // 2), jnp.uint32),
        pltpu.VMEM((CHUNK, D // 2), jnp.uint32),
        pltpu.VMEM((CHUNK, D // 2), jnp.uint32),
        pltpu.VMEM((E_PER_TILE,), jnp.float32),
        pltpu.VMEM_SHARED((N_NODES, D // 2), jnp.uint32),
        pltpu.SemaphoreType.DMA,
        pltpu.SemaphoreType.DMA,
    ],
)
def _decode(z_hbm, ei_hbm, out_hbm, idx_r, idx_c,
            rr0, rc0, rr1, rc1, out_v, z_sp, sem0, sem1):
    sid = lax.axis_index("s")
    wid = sid * NC + lax.axis_index("c")
    base = wid * E_PER_TILE

    @pl.when(sid == 0)
    def _():
        pltpu.sync_copy(z_hbm, z_sp)

    pltpu.sync_copy(ei_hbm.at[0, pl.ds(base, E_PER_TILE)], idx_r)
    pltpu.sync_copy(ei_hbm.at[1, pl.ds(base, E_PER_TILE)], idx_c)
    plsc.subcore_barrier()

    lane = lax.iota(jnp.int32, 16)

    def start(ci, rr, rc, sem):
        o = ci * CHUNK
        pltpu.async_copy(z_sp.at[idx_r.at[pl.ds(o, CHUNK)]], rr, sem)
        pltpu.async_copy(z_sp.at[idx_c.at[pl.ds(o, CHUNK)]], rc, sem)

    def drain(rr, rc, sem):
        pltpu.make_async_copy(z_hbm.at[pl.ds(0, CHUNK)], rr, sem).wait()
        pltpu.make_async_copy(z_hbm.at[pl.ds(0, CHUNK)], rc, sem).wait()

    def compute(off, rr, rc):
        def group_body(g, c2):
            e0 = g * 16

            def _one_quad(q, v):
                ss = []
                for u in range(4):
                    ec = e0 + 4 * q + u
                    p = []
                    for k in range(D // 32):
                        rlo, rhi = _halves(rr[ec, pl.ds(16 * k, 16)])
                        clo, chi = _halves(rc[ec, pl.ds(16 * k, 16)])
                        p.append(rlo * clo)
                        p.append(rhi * chi)
                    ss.append(((p[0] + p[1]) + (p[2] + p[3]))
                              + ((p[4] + p[5]) + (p[6] + p[7])))
                f = [s + _shuffle(s, lane ^ 8) for s in ss]
                m01 = jnp.where(lane < 8, f[0], f[1])
                m23 = jnp.where(lane < 8, f[2], f[3])
                m01 = m01 + _shuffle(m01, lane ^ 4)
                m23 = m23 + _shuffle(m23, lane ^ 4)
                mall = jnp.where((lane & 4) == 0, m01, m23)
                mall = mall + _shuffle(mall, lane ^ 2)
                mall = mall + _shuffle(mall, lane ^ 1)
                lu = lane & 3
                t = _shuffle(mall, ((lu & 1) << 3) | ((lu & 2) << 1))
                return jnp.where((lane >> 2) == q, t, v)

            v = lax.fori_loop(0, 4, _one_quad, jnp.zeros((16,), jnp.float32))
            out_v[pl.ds(off + e0, 16)] = 1.0 / (1.0 + jnp.exp(-v))
            return c2

        lax.fori_loop(0, GROUPS, group_body, 0)

    start(0, rr0, rc0, sem0)

    def pipe_body(i, carry):
        ci = 2 * i
        start(ci + 1, rr1, rc1, sem1)
        drain(rr0, rc0, sem0)
        compute(ci * CHUNK, rr0, rc0)
        start(ci + 2, rr0, rc0, sem0)
        drain(rr1, rc1, sem1)
        compute((ci + 1) * CHUNK, rr1, rc1)
        return carry

    lax.fori_loop(0, (N_CHUNKS - 1) // 2, pipe_body, 0)
    drain(rr0, rc0, sem0)
    compute((N_CHUNKS - 1) * CHUNK, rr0, rc0)
    pltpu.sync_copy(out_v, out_hbm.at[pl.ds(base, E_PER_TILE)])


def _round_bf16_bits(u):
    return (u + jnp.uint32(0x7FFF) + ((u >> 16) & jnp.uint32(1))) >> 16


def kernel(z, edge_index):
    u = lax.bitcast_convert_type(z, jnp.uint32)
    ua = _round_bf16_bits(u[:, : D // 2])
    ub = _round_bf16_bits(u[:, D // 2:])
    zp = ua | (ub << 16)
    return _decode(zp, edge_index.astype(jnp.int32))


# speedup vs baseline: 14.0820x; 1.0007x over previous
import functools

import jax
import jax.numpy as jnp
from jax import lax
from jax.experimental import pallas as pl
from jax.experimental.pallas import tpu as pltpu
from jax.experimental.pallas import tpu_sc as plsc

N_NODES = 10000
N_EDGES = 320000
D = 128

NC = 2
NS = 16
NW = NC * NS
E_PER_TILE = N_EDGES // NW
CHUNK = 80
N_CHUNKS = E_PER_TILE // CHUNK
GROUPS = CHUNK // 16

_mesh = plsc.VectorSubcoreMesh(core_axis_name="c", subcore_axis_name="s")

_GATHER_DIMS = lax.GatherDimensionNumbers(
    offset_dims=(), collapsed_slice_dims=(0,), start_index_map=(0,))


def _shuffle(v, perm):
    return lax.gather(v, perm[:, None], _GATHER_DIMS, slice_sizes=(1,),
                      mode=lax.GatherScatterMode.PROMISE_IN_BOUNDS)


def _halves(u):
    lo = plsc.bitcast(u << jnp.uint32(16), jnp.float32)
    hi = plsc.bitcast(u, jnp.float32)
    return lo, hi


@functools.partial(
    pl.kernel,
    mesh=_mesh,
    compiler_params=pltpu.CompilerParams(needs_layout_passes=False,
                                         use_tc_tiling_on_sc=False),
    out_type=jax.ShapeDtypeStruct((N_EDGES,), jnp.float32),
    scratch_types=[
        pltpu.VMEM((E_PER_TILE,), jnp.int32),
        pltpu.VMEM((E_PER_TILE,), jnp.int32),
        pltpu.VMEM((CHUNK, D // 2), jnp.uint32),
        pltpu.VMEM((CHUNK, D // 2), jnp.uint32),
        pltpu.VMEM((CHUNK, D // 2), jnp.uint32),
        pltpu.VMEM((CHUNK, D // 2), jnp.uint32),
        pltpu.VMEM((E_PER_TILE,), jnp.float32),
        pltpu.VMEM_SHARED((N_NODES, D // 2), jnp.uint32),
        pltpu.SemaphoreType.DMA,
        pltpu.SemaphoreType.DMA,
    ],
)
def _decode(z_hbm, ei_hbm, out_hbm, idx_r, idx_c,
            rr0, rc0, rr1, rc1, out_v, z_sp, sem0, sem1):
    sid = lax.axis_index("s")
    wid = sid * NC + lax.axis_index("c")
    base = wid * E_PER_TILE

    @pl.when(sid == 0)
    def _():
        pltpu.sync_copy(z_hbm, z_sp)

    pltpu.sync_copy(ei_hbm.at[0, pl.ds(base, E_PER_TILE)], idx_r)
    pltpu.sync_copy(ei_hbm.at[1, pl.ds(base, E_PER_TILE)], idx_c)
    plsc.subcore_barrier()

    lane = lax.iota(jnp.int32, 16)

    def start(ci, rr, rc, sem):
        o = ci * CHUNK
        pltpu.async_copy(z_sp.at[idx_r.at[pl.ds(o, CHUNK)]], rr, sem)
        pltpu.async_copy(z_sp.at[idx_c.at[pl.ds(o, CHUNK)]], rc, sem)

    def drain(rr, rc, sem):
        pltpu.make_async_copy(z_hbm.at[pl.ds(0, CHUNK)], rr, sem).wait()
        pltpu.make_async_copy(z_hbm.at[pl.ds(0, CHUNK)], rc, sem).wait()

    def compute(off, rr, rc):
        @plsc.parallel_loop(0, GROUPS)
        def group_body(g):
            e0 = g * 16

            @plsc.parallel_loop(0, 4, carry=jnp.zeros((16,), jnp.float32))
            def v(q, v):
                ss = []
                for u in range(4):
                    ec = e0 + 4 * q + u
                    p = []
                    for k in range(D // 32):
                        rlo, rhi = _halves(rr[ec, pl.ds(16 * k, 16)])
                        clo, chi = _halves(rc[ec, pl.ds(16 * k, 16)])
                        p.append(rlo * clo)
                        p.append(rhi * chi)
                    ss.append(((p[0] + p[1]) + (p[2] + p[3]))
                              + ((p[4] + p[5]) + (p[6] + p[7])))
                f = [s + _shuffle(s, lane ^ 8) for s in ss]
                m01 = jnp.where(lane < 8, f[0], f[1])
                m23 = jnp.where(lane < 8, f[2], f[3])
                m01 = m01 + _shuffle(m01, lane ^ 4)
                m23 = m23 + _shuffle(m23, lane ^ 4)
                mall = jnp.where((lane & 4) == 0, m01, m23)
                mall = mall + _shuffle(mall, lane ^ 2)
                mall = mall + _shuffle(mall, lane ^ 1)
                lu = lane & 3
                t = _shuffle(mall, ((lu & 1) << 3) | ((lu & 2) << 1))
                return jnp.where((lane >> 2) == q, t, v)

            out_v[pl.ds(off + e0, 16)] = 1.0 / (1.0 + jnp.exp(-v))

    start(0, rr0, rc0, sem0)

    def pipe_body(i, carry):
        ci = 2 * i
        start(ci + 1, rr1, rc1, sem1)
        drain(rr0, rc0, sem0)
        compute(ci * CHUNK, rr0, rc0)
        start(ci + 2, rr0, rc0, sem0)
        drain(rr1, rc1, sem1)
        compute((ci + 1) * CHUNK, rr1, rc1)
        return carry

    lax.fori_loop(0, (N_CHUNKS - 1) // 2, pipe_body, 0)
    drain(rr0, rc0, sem0)
    compute((N_CHUNKS - 1) * CHUNK, rr0, rc0)
    pltpu.sync_copy(out_v, out_hbm.at[pl.ds(base, E_PER_TILE)])


def _round_bf16_bits(u):
    return (u + jnp.uint32(0x7FFF) + ((u >> 16) & jnp.uint32(1))) >> 16


def kernel(z, edge_index):
    u = lax.bitcast_convert_type(z, jnp.uint32)
    ua = _round_bf16_bits(u[:, : D // 2])
    ub = _round_bf16_bits(u[:, D // 2:])
    zp = ua | (ub << 16)
    return _decode(zp, edge_index.astype(jnp.int32))
